# baseline (speedup 1.0000x reference)
import jax
import jax.numpy as jnp
from jax.experimental import pallas as pl
from jax.experimental.pallas import tpu as pltpu

H = 32
W_ = 32
P = H * W_
CIN = 64
COUT = 64
KK = 9
BB = 16
PB = 256
QW = 768
NJ = P // PB
QBASE = (0, 0, 256, 256)


def _body(x_ref, w_ref, idxc_ref, out_ref, s_ref, xb_ref, x1_ref, xc_ref):
    @pl.when(pl.program_id(0) == 0)
    def _build_onehot():
        iq = jax.lax.broadcasted_iota(jnp.int32, (QW, KK * PB), 0)
        for j in range(NJ):
            cols = idxc_ref[j]
            s_ref[j] = jnp.where(
                iq + QBASE[j] == cols, 1.0, 0.0).astype(jnp.bfloat16)

    xb_ref[...] = x_ref[...].reshape(BB * CIN, P)
    for j in range(NJ):
        x1_ref[...] = jnp.dot(
            xb_ref[:, pl.ds(QBASE[j], QW)], s_ref[j],
            preferred_element_type=jnp.float32,
        ).astype(jnp.bfloat16)
        for b in range(BB):
            for k in range(KK):
                xc_ref[pl.ds(k * CIN, CIN), pl.ds(b * PB, PB)] = (
                    x1_ref[pl.ds(b * CIN, CIN), pl.ds(k * PB, PB)])
        oj = jnp.dot(w_ref[...], xc_ref[...],
                     preferred_element_type=jnp.float32)
        for b in range(BB):
            out_ref[b, :, pl.ds(j * PB, PB)] = oj[:, b * PB:(b + 1) * PB]


def kernel(x, W, sample_idx):
    B = x.shape[0]
    w2 = jnp.transpose(W.reshape(COUT, CIN, KK), (0, 2, 1)).reshape(
        COUT, KK * CIN).astype(jnp.bfloat16)
    idxc = jnp.transpose(
        sample_idx.reshape(NJ, PB, KK), (0, 2, 1)).reshape(NJ, 1, KK * PB)

    xb16 = x.reshape(B, CIN, P).astype(jnp.bfloat16)
    out = pl.pallas_call(
        _body,
        grid=(B // BB,),
        in_specs=[
            pl.BlockSpec((BB, CIN, P), lambda b: (b, 0, 0)),
            pl.BlockSpec((COUT, KK * CIN), lambda b: (0, 0)),
            pl.BlockSpec((NJ, 1, KK * PB), lambda b: (0, 0, 0)),
        ],
        out_specs=pl.BlockSpec((BB, COUT, P), lambda b: (b, 0, 0)),
        out_shape=jax.ShapeDtypeStruct((B, COUT, P), jnp.float32),
        scratch_shapes=[
            pltpu.VMEM((NJ, QW, KK * PB), jnp.bfloat16),
            pltpu.VMEM((BB * CIN, P), jnp.bfloat16),
            pltpu.VMEM((BB * CIN, KK * PB), jnp.bfloat16),
            pltpu.VMEM((KK * CIN, BB * PB), jnp.bfloat16),
        ],
    )(xb16, w2, idxc)
    return out.reshape(B, COUT, H, W_)


# speedup vs baseline: 1.1334x; 1.1334x over previous
import jax
import jax.numpy as jnp
from jax.experimental import pallas as pl
from jax.experimental.pallas import tpu as pltpu

H = 32
W_ = 32
P = H * W_
CIN = 64
COUT = 64
KK = 9
BB = 16
PBS = (240, 240, 240, 240, 64)
P0S = (0, 240, 480, 720, 960)
QS = (0, 104, 344, 512, 512)
QW = 512
NJ = len(PBS)
NCOL = KK * 256


def _body(x_ref, w_ref, idxc_ref, out_ref, s_ref, xb_ref, x1_ref, xc_ref):
    @pl.when(pl.program_id(0) == 0)
    def _build_onehot():
        iq = jax.lax.broadcasted_iota(jnp.int32, (QW, NCOL), 0)
        for j in range(NJ):
            cols = idxc_ref[j]
            s_ref[j] = jnp.where(
                iq + QS[j] == cols, 1.0, 0.0).astype(jnp.bfloat16)

    xb_ref[...] = x_ref[...].reshape(BB * CIN, P).astype(jnp.bfloat16)
    for j in range(NJ):
        pb = PBS[j]
        ncol = ((KK * pb + 255) // 256) * 256
        x1_ref[:, pl.ds(0, ncol)] = jnp.dot(
            xb_ref[:, pl.ds(QS[j], QW)], s_ref[j, :, pl.ds(0, ncol)],
            preferred_element_type=jnp.float32,
        ).astype(jnp.bfloat16)
        for b in range(BB):
            for k in range(KK):
                xc_ref[pl.ds(k * CIN, CIN), pl.ds(b * pb, pb)] = (
                    x1_ref[pl.ds(b * CIN, CIN), pl.ds(k * pb, pb)])
        oj = jnp.dot(w_ref[...], xc_ref[:, pl.ds(0, BB * pb)],
                     preferred_element_type=jnp.float32)
        for b in range(BB):
            out_ref[b, :, pl.ds(P0S[j], pb)] = oj[:, b * pb:(b + 1) * pb]


def kernel(x, W, sample_idx):
    B = x.shape[0]
    w2 = jnp.transpose(W.reshape(COUT, CIN, KK), (0, 2, 1)).reshape(
        COUT, KK * CIN).astype(jnp.bfloat16)
    si = sample_idx.reshape(P, KK)
    rows = []
    for j in range(NJ):
        blk = jnp.transpose(si[P0S[j]:P0S[j] + PBS[j], :], (1, 0)).reshape(-1)
        rows.append(jnp.pad(blk, (0, NCOL - blk.shape[0]),
                            constant_values=-1))
    idxc = jnp.stack(rows).reshape(NJ, 1, NCOL)

    out = pl.pallas_call(
        _body,
        grid=(B // BB,),
        in_specs=[
            pl.BlockSpec((BB, CIN, P), lambda b: (b, 0, 0)),
            pl.BlockSpec((COUT, KK * CIN), lambda b: (0, 0)),
            pl.BlockSpec((NJ, 1, NCOL), lambda b: (0, 0, 0)),
        ],
        out_specs=pl.BlockSpec((BB, COUT, P), lambda b: (b, 0, 0)),
        out_shape=jax.ShapeDtypeStruct((B, COUT, P), jnp.float32),
        scratch_shapes=[
            pltpu.VMEM((NJ, QW, NCOL), jnp.bfloat16),
            pltpu.VMEM((BB * CIN, P), jnp.bfloat16),
            pltpu.VMEM((BB * CIN, NCOL), jnp.bfloat16),
            pltpu.VMEM((KK * CIN, BB * 256), jnp.bfloat16),
        ],
    )(x.reshape(B, CIN, P), w2, idxc)
    return out.reshape(B, COUT, H, W_)


# speedup vs baseline: 1.1378x; 1.0039x over previous
import jax
import jax.numpy as jnp
from jax.experimental import pallas as pl
from jax.experimental.pallas import tpu as pltpu

H = 32
W_ = 32
P = H * W_
CIN = 64
COUT = 64
KK = 9
BB = 16
PBS = (240, 240, 240, 240, 64)
P0S = (0, 240, 480, 720, 960)
QS = (0, 104, 344, 512, 512)
QW = 512
NJ = len(PBS)
PADS = (256, 256, 256, 256, 128)
NCOL = KK * 256


def _body(x_ref, w_ref, idxc_ref, out_ref, s_ref, xb_ref, x1_ref, xc_ref):
    @pl.when(pl.program_id(0) == 0)
    def _build_onehot():
        iq = jax.lax.broadcasted_iota(jnp.int32, (QW, NCOL), 0)
        for j in range(NJ):
            cols = idxc_ref[j]
            s_ref[j] = jnp.where(
                iq + QS[j] == cols, 1.0, 0.0).astype(jnp.bfloat16)

    xb_ref[...] = x_ref[...].reshape(BB * CIN, P).astype(jnp.bfloat16)
    for j in range(NJ):
        pb = PBS[j]
        pad = PADS[j]
        ncol = KK * pad
        x1_ref[:, pl.ds(0, ncol)] = jnp.dot(
            xb_ref[:, pl.ds(QS[j], QW)], s_ref[j, :, pl.ds(0, ncol)],
            preferred_element_type=jnp.float32,
        ).astype(jnp.bfloat16)
        for b in range(BB):
            for k in range(KK):
                xc_ref[pl.ds(k * CIN, CIN), pl.ds(b * pad, pad)] = (
                    x1_ref[pl.ds(b * CIN, CIN), pl.ds(k * pad, pad)])
        oj = jnp.dot(w_ref[...], xc_ref[:, pl.ds(0, BB * pad)],
                     preferred_element_type=jnp.float32)
        for b in range(BB):
            out_ref[b, :, pl.ds(P0S[j], pb)] = oj[:, b * pad:b * pad + pb]


def kernel(x, W, sample_idx):
    B = x.shape[0]
    w2 = jnp.transpose(W.reshape(COUT, CIN, KK), (0, 2, 1)).reshape(
        COUT, KK * CIN).astype(jnp.bfloat16)
    si = sample_idx.reshape(P, KK)
    rows = []
    for j in range(NJ):
        blk = jnp.transpose(si[P0S[j]:P0S[j] + PBS[j], :], (1, 0))
        blk = jnp.pad(blk, ((0, 0), (0, PADS[j] - PBS[j])),
                      constant_values=-1).reshape(-1)
        rows.append(jnp.pad(blk, (0, NCOL - blk.shape[0]),
                            constant_values=-1))
    idxc = jnp.stack(rows).reshape(NJ, 1, NCOL)

    out = pl.pallas_call(
        _body,
        grid=(B // BB,),
        in_specs=[
            pl.BlockSpec((BB, CIN, P), lambda b: (b, 0, 0)),
            pl.BlockSpec((COUT, KK * CIN), lambda b: (0, 0)),
            pl.BlockSpec((NJ, 1, NCOL), lambda b: (0, 0, 0)),
        ],
        out_specs=pl.BlockSpec((BB, COUT, P), lambda b: (b, 0, 0)),
        out_shape=jax.ShapeDtypeStruct((B, COUT, P), jnp.float32),
        scratch_shapes=[
            pltpu.VMEM((NJ, QW, NCOL), jnp.bfloat16),
            pltpu.VMEM((BB * CIN, P), jnp.bfloat16),
            pltpu.VMEM((BB * CIN, NCOL), jnp.bfloat16),
            pltpu.VMEM((KK * CIN, BB * 256), jnp.bfloat16),
        ],
    )(x.reshape(B, CIN, P), w2, idxc)
    return out.reshape(B, COUT, H, W_)


# speedup vs baseline: 1.1422x; 1.0039x over previous
import jax
import jax.numpy as jnp
from jax.experimental import pallas as pl
from jax.experimental.pallas import tpu as pltpu

H = 32
W_ = 32
P = H * W_
CIN = 64
COUT = 64
KK = 9
BB = 16
PBS = (240, 240, 240, 240, 64)
P0S = (0, 240, 480, 720, 960)
QS = (0, 104, 344, 512, 512)
QW = 512
NJ = len(PBS)
PADS = (256, 256, 256, 256, 128)
NCOL = KK * 256


def _body(x_ref, w_ref, idxc_ref, out_ref, s_ref, xb_ref, x1_ref, xc_ref):
    @pl.when(pl.program_id(0) == 0)
    def _build_onehot():
        iq = jax.lax.broadcasted_iota(jnp.int32, (QW, NCOL), 0)
        for j in range(NJ):
            cols = idxc_ref[j]
            s_ref[j] = jnp.where(
                iq + QS[j] == cols, 1.0, 0.0).astype(jnp.bfloat16)

    xb_ref[...] = x_ref[...].reshape(BB * CIN, P).astype(jnp.bfloat16)
    for j in range(NJ + 1):
        if j < NJ:
            ncol = KK * PADS[j]
            x1_ref[j % 2, :, pl.ds(0, ncol)] = jnp.dot(
                xb_ref[:, pl.ds(QS[j], QW)], s_ref[j, :, pl.ds(0, ncol)],
                preferred_element_type=jnp.float32,
            ).astype(jnp.bfloat16)
        if j >= 1:
            jj = j - 1
            pb, pad = PBS[jj], PADS[jj]
            for b in range(BB):
                for k in range(KK):
                    xc_ref[pl.ds(k * CIN, CIN), pl.ds(b * pad, pad)] = (
                        x1_ref[jj % 2, pl.ds(b * CIN, CIN),
                               pl.ds(k * pad, pad)])
            oj = jnp.dot(w_ref[...], xc_ref[:, pl.ds(0, BB * pad)],
                         preferred_element_type=jnp.float32)
            for b in range(BB):
                out_ref[b, :, pl.ds(P0S[jj], pb)] = oj[:, b * pad:b * pad + pb]


def kernel(x, W, sample_idx):
    B = x.shape[0]
    w2 = jnp.transpose(W.reshape(COUT, CIN, KK), (0, 2, 1)).reshape(
        COUT, KK * CIN).astype(jnp.bfloat16)
    si = sample_idx.reshape(P, KK)
    rows = []
    for j in range(NJ):
        blk = jnp.transpose(si[P0S[j]:P0S[j] + PBS[j], :], (1, 0))
        blk = jnp.pad(blk, ((0, 0), (0, PADS[j] - PBS[j])),
                      constant_values=-1).reshape(-1)
        rows.append(jnp.pad(blk, (0, NCOL - blk.shape[0]),
                            constant_values=-1))
    idxc = jnp.stack(rows).reshape(NJ, 1, NCOL)

    out = pl.pallas_call(
        _body,
        grid=(B // BB,),
        in_specs=[
            pl.BlockSpec((BB, CIN, P), lambda b: (b, 0, 0)),
            pl.BlockSpec((COUT, KK * CIN), lambda b: (0, 0)),
            pl.BlockSpec((NJ, 1, NCOL), lambda b: (0, 0, 0)),
        ],
        out_specs=pl.BlockSpec((BB, COUT, P), lambda b: (b, 0, 0)),
        out_shape=jax.ShapeDtypeStruct((B, COUT, P), jnp.float32),
        scratch_shapes=[
            pltpu.VMEM((NJ, QW, NCOL), jnp.bfloat16),
            pltpu.VMEM((BB * CIN, P), jnp.bfloat16),
            pltpu.VMEM((2, BB * CIN, NCOL), jnp.bfloat16),
            pltpu.VMEM((KK * CIN, BB * 256), jnp.bfloat16),
        ],
    )(x.reshape(B, CIN, P), w2, idxc)
    return out.reshape(B, COUT, H, W_)
